# baseline (speedup 1.0000x reference)
import functools

import jax
import jax.numpy as jnp
from jax import lax
from jax.experimental import pallas as pl
from jax.experimental.pallas import tpu as pltpu
from jax.experimental.pallas import tpu_sc as plsc

B, C, N_IN, N_OUT, K = 4, 128, 50000, 12500, 8
NC, NS = 2, 16
NW = NC * NS
JW = 392
JB = 32
NBLK = 13
NROW = JB * B
NBST = 48
J_PAD = 12512
J_LAST = N_OUT - JB
INV_K = 1.0 / K


def _sc_body(xt_hbm, nb_hbm, out_hbm, nbw_v, idx_v, acc_v, stage_v,
             gsem0, gsem1, nsem0, nsem1, osem0, osem1):
    wid = lax.axis_index("s") * NC + lax.axis_index("c")
    j0w = wid * JW
    gsems = (gsem0, gsem1)
    nsems = (nsem0, nsem1)
    osems = (osem0, osem1)

    lane = lax.iota(jnp.int32, 16) * B
    poss = [lane + jnp.int32(h * 16 * B + b)
            for h in range(JB // 16) for b in range(B)]
    offs = [jnp.int32(b * N_IN) for b in range(B)]

    def j0_of(i):
        return lax.min(j0w + i * JB, J_LAST)

    def j0a_of(i):
        return (j0_of(i) // 8) * 8

    def issue_nb(i, buf):
        j0a = j0a_of(i)
        return [pltpu.async_copy(
            nb_hbm.at[pl.ds(k * J_PAD + j0a, NBST)],
            nbw_v.at[buf, k], nsems[buf]) for k in range(K)]

    def build_idx(i, buf):
        d = j0_of(i) - j0a_of(i)
        for k in range(K):
            for h in range(JB // 16):
                nbv = nbw_v[buf, k, pl.ds(d + h * 16, 16)]
                for b in range(B):
                    plsc.store_scatter(idx_v.at[buf, k],
                                       [poss[h * B + b]], nbv + offs[b])

    def zero_acc(buf):
        def zb(r, carry):
            for t in range(C // 16):
                acc_v[buf, r, pl.ds(t * 16, 16)] = jnp.zeros((16,), jnp.float32)
            return carry
        lax.fori_loop(0, NROW, zb, 0)

    ZERO16 = jnp.zeros((16,), jnp.float32)

    def issue_gathers(ibuf, abuf):
        return [pltpu.async_copy(
            xt_hbm.at[idx_v.at[ibuf, k]],
            acc_v.at[abuf], gsems[ibuf], add=True) for k in range(K)]

    def scale_and_send(i, abuf, sbuf):
        def sb(r, carry):
            for t in range(C // 16):
                v = acc_v[abuf, r, pl.ds(t * 16, 16)]
                stage_v[sbuf, pl.ds(r * C + t * 16, 16)] = v * jnp.float32(INV_K)
                acc_v[abuf, r, pl.ds(t * 16, 16)] = ZERO16
            return carry
        lax.fori_loop(0, NROW, sb, 0)
        j0 = j0_of(i)
        return pltpu.async_copy(stage_v.at[sbuf],
                                out_hbm.at[pl.ds(j0 * B * C, NROW * C)],
                                osems[sbuf])

    for a in range(3):
        zero_acc(a)
    g_descs = {}
    out_descs = {}
    nb_descs = {0: issue_nb(0, 0)}
    for i in range(NBLK):
        ibuf = i % 2
        for d in nb_descs.pop(i):
            d.wait()
        if i + 1 < NBLK:
            nb_descs[i + 1] = issue_nb(i + 1, 1 - ibuf)
        if i >= 2:
            out_descs.pop(i - 2).wait()
        build_idx(i, ibuf)
        g_descs[i] = issue_gathers(ibuf, i % 3)
        if i >= 1:
            for d in g_descs.pop(i - 1):
                d.wait()
            out_descs[i - 1] = scale_and_send(i - 1, (i - 1) % 3, (i - 1) % 2)
    last = NBLK - 1
    for d in g_descs.pop(last):
        d.wait()
    out_descs[last] = scale_and_send(last, last % 3, last % 2)
    out_descs.pop(last - 1).wait()
    out_descs.pop(last).wait()


@functools.partial(
    pl.kernel,
    out_type=jax.ShapeDtypeStruct((N_OUT * B * C,), jnp.float32),
    mesh=plsc.VectorSubcoreMesh(core_axis_name="c", subcore_axis_name="s"),
    scratch_types=[
        pltpu.VMEM((2, K, NBST), jnp.int32),
        pltpu.VMEM((2, K, NROW), jnp.int32),
        pltpu.VMEM((3, NROW, C), jnp.float32),
        pltpu.VMEM((2, NROW * C), jnp.float32),
        pltpu.SemaphoreType.DMA,
        pltpu.SemaphoreType.DMA,
        pltpu.SemaphoreType.DMA,
        pltpu.SemaphoreType.DMA,
        pltpu.SemaphoreType.DMA,
        pltpu.SemaphoreType.DMA,
    ],
    compiler_params=pltpu.CompilerParams(needs_layout_passes=False),
)
def _avg_pool_sc(xt_hbm, nb_hbm, out_hbm, nbw_v, idx_v, acc_v, stage_v,
                 gsem0, gsem1, nsem0, nsem1, osem0, osem1):
    _sc_body(xt_hbm, nb_hbm, out_hbm, nbw_v, idx_v, acc_v, stage_v,
             gsem0, gsem1, nsem0, nsem1, osem0, osem1)


def kernel(x, neighbours):
    nb = neighbours.astype(jnp.int32)
    nb1d = jnp.pad(nb, ((0, 0), (0, J_PAD - N_OUT))).reshape(-1)
    xt = x.transpose(0, 2, 1).reshape(B * N_IN, C)
    out1d = _avg_pool_sc(xt, nb1d)
    return out1d.reshape(N_OUT, B, C).transpose(1, 2, 0)
